# baseline (speedup 1.0000x reference)
import functools

import jax
import jax.numpy as jnp
from jax import lax
from jax.experimental import pallas as pl
from jax.experimental.pallas import tpu as pltpu
from jax.experimental.pallas import tpu_sc as plsc

K = 100000
Q = 16
F = 64
BK = 5000
NB = K // BK


def _argmin_body(x_ref, audio_ref, dummy_ref, idx_ref, loss_ref, rmin_ref, ridx_ref):
    pid = pl.program_id(0)
    xm = jnp.mean(x_ref[...], axis=1)
    blk = audio_ref[...]
    dotT = lax.dot_general(xm, blk, (((1,), (1,)), ((), ())),
                           preferred_element_type=jnp.float32,
                           precision=lax.Precision.HIGHEST)
    ones = jnp.ones((1, F), jnp.float32)
    rnormT = lax.dot_general(ones, blk * blk, (((1,), (1,)), ((), ())),
                             preferred_element_type=jnp.float32,
                             precision=lax.Precision.HIGHEST)
    scores = rnormT - 2.0 * dotT

    bmin = jnp.min(scores, axis=1, keepdims=True)
    cols = lax.broadcasted_iota(jnp.int32, (Q, BK), 1) + pid * BK
    bidx = jnp.min(jnp.where(scores == bmin, cols, K), axis=1, keepdims=True)

    @pl.when(pid == 0)
    def _():
        rmin_ref[...] = bmin
        ridx_ref[...] = bidx

    @pl.when(pid > 0)
    def _():
        upd = bmin < rmin_ref[...]
        rmin_ref[...] = jnp.where(upd, bmin, rmin_ref[...])
        ridx_ref[...] = jnp.where(upd, bidx, ridx_ref[...])

    @pl.when(pid == NB - 1)
    def _():
        idx_ref[...] = ridx_ref[...]
        loss_ref[...] = jnp.sum(dummy_ref[...], keepdims=True)


_argmin_call = pl.pallas_call(
    _argmin_body,
    grid=(NB,),
    in_specs=[
        pl.BlockSpec((Q, 20, F), lambda i: (0, 0, 0)),
        pl.BlockSpec((BK, F), lambda i: (i, 0)),
        pl.BlockSpec((1, 1), lambda i: (0, 0)),
    ],
    out_specs=[
        pl.BlockSpec((Q, 1), lambda i: (0, 0)),
        pl.BlockSpec((1, 1), lambda i: (0, 0)),
    ],
    out_shape=[
        jax.ShapeDtypeStruct((Q, 1), jnp.int32),
        jax.ShapeDtypeStruct((1, 1), jnp.float32),
    ],
    scratch_shapes=[
        pltpu.VMEM((Q, 1), jnp.float32),
        pltpu.VMEM((Q, 1), jnp.int32),
    ],
)


def _gather_body(idx_sref, pose_ref, out_ref):
    out_ref[...] = pose_ref[...]


_gather_call = pl.pallas_call(
    _gather_body,
    grid_spec=pltpu.PrefetchScalarGridSpec(
        num_scalar_prefetch=1,
        grid=(Q,),
        in_specs=[pl.BlockSpec((1, 20, 32), lambda q, idx_ref: (idx_ref[q], 0, 0))],
        out_specs=pl.BlockSpec((1, 20, 32), lambda q, idx_ref: (q, 0, 0)),
    ),
    out_shape=jax.ShapeDtypeStruct((Q, 20, 32), jnp.float32),
)


@jax.jit
def kernel(x, y, audio, pose, dummy):
    idx2d, loss = _argmin_call(x[0], audio, dummy.reshape(1, 1))
    out = _gather_call(idx2d.reshape(Q), pose)
    return (out, loss[0, 0])


# speedup vs baseline: 25.0181x; 25.0181x over previous
import functools

import jax
import jax.numpy as jnp
from jax import lax
from jax.experimental import pallas as pl
from jax.experimental.pallas import tpu as pltpu

K = 100000
Q = 16
F = 64
T = 20
P = 32


def _argmin_body(xT_ref, audioT_ref, dummy_ref, idx_ref, loss_ref):
    xm = jnp.mean(xT_ref[...], axis=0)
    blkT = audioT_ref[...]
    dotT = lax.dot_general(xm, blkT, (((1,), (0,)), ((), ())),
                           preferred_element_type=jnp.float32,
                           precision=lax.Precision.HIGHEST)
    rn = jnp.zeros((1, K), jnp.float32)
    for c in range(F // 8):
        ch = audioT_ref[8 * c:8 * (c + 1), :]
        rn = rn + jnp.sum(ch * ch, axis=0, keepdims=True)
    scores = rn - 2.0 * dotT

    bmin = jnp.min(scores, axis=1, keepdims=True)
    cols = lax.broadcasted_iota(jnp.int32, (Q, K), 1)
    idx_ref[...] = jnp.min(jnp.where(scores == bmin, cols, K),
                           axis=1, keepdims=True)
    loss_ref[...] = jnp.sum(dummy_ref[...], keepdims=True)


_argmin_call = pl.pallas_call(
    _argmin_body,
    in_specs=[
        pl.BlockSpec((T, Q, F), lambda: (0, 0, 0)),
        pl.BlockSpec((F, K), lambda: (0, 0)),
        pl.BlockSpec((1, 1), lambda: (0, 0)),
    ],
    out_specs=[
        pl.BlockSpec((Q, 1), lambda: (0, 0)),
        pl.BlockSpec((1, 1), lambda: (0, 0)),
    ],
    out_shape=[
        jax.ShapeDtypeStruct((Q, 1), jnp.int32),
        jax.ShapeDtypeStruct((1, 1), jnp.float32),
    ],
)


def _tile_copy(idx_sref, poseT_ref, tiles_ref, sem, q):
    base = pl.multiple_of((idx_sref[q] // 128) * 128, 128)
    return pltpu.make_async_copy(
        poseT_ref.at[:, :, pl.ds(base, 128)],
        tiles_ref.at[pl.ds(q * T, T)],
        sem,
    )


def _gather_body(idx_sref, poseT_ref, outT_ref, tiles_ref, sem):
    for q in range(Q):
        _tile_copy(idx_sref, poseT_ref, tiles_ref, sem, q).start()
    for q in range(Q):
        _tile_copy(idx_sref, poseT_ref, tiles_ref, sem, q).wait()
        lane = idx_sref[q] % 128
        tile = tiles_ref[q * T:(q + 1) * T]
        rolled = pltpu.roll(tile, (128 - lane) % 128, 2)
        outT_ref[:, :, q:q + 1] = rolled[:, :, 0:1]


_gather_call = pl.pallas_call(
    _gather_body,
    grid_spec=pltpu.PrefetchScalarGridSpec(
        num_scalar_prefetch=1,
        grid=(1,),
        in_specs=[pl.BlockSpec(memory_space=pltpu.MemorySpace.HBM)],
        out_specs=pl.BlockSpec((T, P, Q), lambda i, r: (0, 0, 0)),
        scratch_shapes=[
            pltpu.VMEM((Q * T, P, 128), jnp.float32),
            pltpu.SemaphoreType.DMA,
        ],
    ),
    out_shape=jax.ShapeDtypeStruct((T, P, Q), jnp.float32),
)


@jax.jit
def kernel(x, y, audio, pose, dummy):
    xT = lax.transpose(x[0], (1, 0, 2))
    audioT = lax.transpose(audio, (1, 0))
    poseT = lax.transpose(pose, (1, 2, 0))
    idx2d, loss = _argmin_call(xT, audioT, dummy.reshape(1, 1))
    outT = _gather_call(idx2d.reshape(Q), poseT)
    out = lax.transpose(outT, (2, 0, 1))
    return (out, loss[0, 0])
